# baseline (speedup 1.0000x reference)
import functools

import jax
import jax.numpy as jnp
from jax import lax
from jax.experimental import pallas as pl
from jax.experimental.pallas import tpu as pltpu
from jax.experimental.pallas import tpu_sc as plsc

N = 10000
E = 320000
D = 128

NC = 2
NS = 16
LANES = 16

NW = NC * NS
EDGES_PER_W = E // NW
CHUNK = 80
NCHUNKS = 128
NCHUNKS_PAD = 136
NPAIRS = NCHUNKS // 2
NROW_CHUNKS = N // CHUNK

_DNUMS = lax.GatherDimensionNumbers(
    offset_dims=(), collapsed_slice_dims=(0,), start_index_map=(0,))


def _sc_body(x_hbm, rows_hbm, cols_hbm, vals_hbm, out_hbm,
             colsA, rowsA, valsA, colsB, rowsB, valsB,
             gbuf0, gbuf1, agg,
             gsem0, gsem1, ssem0, ssem1, isemA, isemB):
    c = lax.axis_index("c")
    s = lax.axis_index("s")
    w = c * NS + s

    zero16 = jnp.zeros((LANES,), jnp.float32)

    def _zrow(r, carry):
        for k in range(D // LANES):
            gbuf0[r, pl.ds(k * LANES, LANES)] = zero16
        return carry

    lax.fori_loop(0, CHUNK, _zrow, 0)

    for i in range((NROW_CHUNKS + NS - 1) // NS):
        cid = s + i * NS

        @pl.when(cid < NROW_CHUNKS)
        def _zero_chunk():
            r0 = pl.multiple_of(cid * CHUNK, CHUNK)
            pltpu.sync_copy(gbuf0, agg.at[pl.ds(r0, CHUNK)])

    plsc.subcore_barrier()

    def i_start(p, cb, rb, vb, sem):
        pltpu.async_copy(cols_hbm.at[w, pl.ds(p * 2, 2)], cb, sem)
        pltpu.async_copy(rows_hbm.at[w, pl.ds(p * 2, 2)], rb, sem)
        pltpu.async_copy(vals_hbm.at[w, pl.ds(p * 2, 2)], vb, sem)

    def i_wait(p, cb, rb, vb, sem):
        pltpu.make_async_copy(cols_hbm.at[w, pl.ds(p * 2, 2)], cb, sem).wait()
        pltpu.make_async_copy(rows_hbm.at[w, pl.ds(p * 2, 2)], rb, sem).wait()
        pltpu.make_async_copy(vals_hbm.at[w, pl.ds(p * 2, 2)], vb, sem).wait()

    def g_start(cb, lj, buf, sem):
        pltpu.async_copy(x_hbm.at[cb.at[lj]], buf, sem)

    def g_wait(cb, lj, buf, sem):
        pltpu.make_async_copy(x_hbm.at[cb.at[lj]], buf, sem).wait()

    def s_start(rb, lj, buf, sem):
        pltpu.async_copy(buf, agg.at[rb.at[lj]], sem, add=True)

    def s_wait(rb, lj, buf, sem):
        pltpu.make_async_copy(buf, agg.at[rb.at[lj]], sem).wait()

    def scale(buf, vb, lj):
        def grp(g, carry):
            vv = vb[lj, pl.ds(g * LANES, LANES)]
            for jj in range(LANES):
                bc = lax.gather(
                    vv, jnp.full((LANES, 1), jj, jnp.int32), _DNUMS, (1,),
                    mode=lax.GatherScatterMode.PROMISE_IN_BOUNDS)
                r = g * LANES + jj
                for k in range(D // LANES):
                    sl = pl.ds(k * LANES, LANES)
                    buf[r, sl] = buf[r, sl] * bc
            return carry

        lax.fori_loop(0, CHUNK // LANES, grp, 0)

    i_start(0, colsA, rowsA, valsA, isemA)
    i_wait(0, colsA, rowsA, valsA, isemA)
    g_start(colsA, 0, gbuf0, gsem0)
    i_start(1, colsB, rowsB, valsB, isemB)

    def _quad(t, carry):
        p0 = 2 * t

        g_start(colsA, 1, gbuf1, gsem1)
        g_wait(colsA, 0, gbuf0, gsem0)
        scale(gbuf0, valsA, 0)
        s_start(rowsA, 0, gbuf0, ssem0)

        i_wait(p0 + 1, colsB, rowsB, valsB, isemB)
        s_wait(rowsA, 0, gbuf0, ssem0)
        g_start(colsB, 0, gbuf0, gsem0)

        g_wait(colsA, 1, gbuf1, gsem1)
        scale(gbuf1, valsA, 1)
        s_start(rowsA, 1, gbuf1, ssem1)
        s_wait(rowsA, 1, gbuf1, ssem1)
        i_start(p0 + 2, colsA, rowsA, valsA, isemA)

        g_start(colsB, 1, gbuf1, gsem1)
        g_wait(colsB, 0, gbuf0, gsem0)
        scale(gbuf0, valsB, 0)
        s_start(rowsB, 0, gbuf0, ssem0)

        i_wait(p0 + 2, colsA, rowsA, valsA, isemA)
        s_wait(rowsB, 0, gbuf0, ssem0)
        g_start(colsA, 0, gbuf0, gsem0)

        g_wait(colsB, 1, gbuf1, gsem1)
        scale(gbuf1, valsB, 1)
        s_start(rowsB, 1, gbuf1, ssem1)
        s_wait(rowsB, 1, gbuf1, ssem1)
        i_start(p0 + 3, colsB, rowsB, valsB, isemB)

        return carry

    lax.fori_loop(0, NPAIRS // 2, _quad, 0)

    g_wait(colsA, 0, gbuf0, gsem0)
    i_wait(NPAIRS + 1, colsB, rowsB, valsB, isemB)

    plsc.subcore_barrier()

    for i in range((NROW_CHUNKS + NS - 1) // NS):
        cid = s + i * NS

        @pl.when(cid < NROW_CHUNKS)
        def _copy_chunk():
            r0 = pl.multiple_of(cid * CHUNK, CHUNK)
            pltpu.sync_copy(agg.at[pl.ds(r0, CHUNK)],
                            out_hbm.at[c, pl.ds(r0, CHUNK)])


_sc_segment_sum = functools.partial(
    pl.kernel,
    out_type=jax.ShapeDtypeStruct((NC, N, D), jnp.float32),
    mesh=plsc.VectorSubcoreMesh(core_axis_name="c", subcore_axis_name="s"),
    scratch_types=[
        pltpu.VMEM((2, CHUNK), jnp.int32),
        pltpu.VMEM((2, CHUNK), jnp.int32),
        pltpu.VMEM((2, CHUNK), jnp.float32),
        pltpu.VMEM((2, CHUNK), jnp.int32),
        pltpu.VMEM((2, CHUNK), jnp.int32),
        pltpu.VMEM((2, CHUNK), jnp.float32),
        pltpu.VMEM((CHUNK, D), jnp.float32),
        pltpu.VMEM((CHUNK, D), jnp.float32),
        pltpu.VMEM_SHARED((N, D), jnp.float32),
        pltpu.SemaphoreType.DMA,
        pltpu.SemaphoreType.DMA,
        pltpu.SemaphoreType.DMA,
        pltpu.SemaphoreType.DMA,
        pltpu.SemaphoreType.DMA,
        pltpu.SemaphoreType.DMA,
    ],
)(_sc_body)


BLK = 1000


def _tc_linear_body(p0_ref, p1_ref, wt_ref, b_ref, o_ref):
    acc = p0_ref[...] + p1_ref[...]
    o_ref[...] = (
        jnp.dot(acc, wt_ref[...], preferred_element_type=jnp.float32)
        + b_ref[...]
    )


def _tc_linear(p0, p1, wt, b2):
    return pl.pallas_call(
        _tc_linear_body,
        grid=(N // BLK,),
        in_specs=[
            pl.BlockSpec((BLK, D), lambda i: (i, 0)),
            pl.BlockSpec((BLK, D), lambda i: (i, 0)),
            pl.BlockSpec((D, D), lambda i: (0, 0)),
            pl.BlockSpec((1, D), lambda i: (0, 0)),
        ],
        out_specs=pl.BlockSpec((BLK, D), lambda i: (i, 0)),
        out_shape=jax.ShapeDtypeStruct((N, D), jnp.float32),
    )(p0, p1, wt, b2)


def kernel(X, L_rows, L_cols, L_vals, W, b):
    pad = NCHUNKS_PAD * CHUNK - EDGES_PER_W
    rows3 = jnp.pad(L_rows.reshape(NW, EDGES_PER_W),
                    ((0, 0), (0, pad))).reshape(NW, NCHUNKS_PAD, CHUNK)
    cols3 = jnp.pad(L_cols.reshape(NW, EDGES_PER_W),
                    ((0, 0), (0, pad))).reshape(NW, NCHUNKS_PAD, CHUNK)
    vals3 = jnp.pad(L_vals.reshape(NW, EDGES_PER_W),
                    ((0, 0), (0, pad))).reshape(NW, NCHUNKS_PAD, CHUNK)
    partials = _sc_segment_sum(X, rows3, cols3, vals3)
    return _tc_linear(partials[0], partials[1], W.T, b.reshape(1, D))
